# baseline (speedup 1.0000x reference)
import functools

import jax
import jax.numpy as jnp
from jax import lax
from jax.experimental import pallas as pl
from jax.experimental.pallas import tpu as pltpu
from jax.experimental.pallas import tpu_sc as plsc

N = 10000
E = 160000
D = 256
DH = 128

NC = 2
NT = 16
NPAD = 10112
RPT = NPAD // NT
NPAD_DEG = 10240
RPT_DEG = NPAD_DEG // NT
EPAD_DEG = 163840
EPAD = 165888
CHUNK = 128
NBUF = 3

RB = 1000
NRB = N // RB


@functools.cache
def _sc_kernels():
    mesh = plsc.VectorSubcoreMesh(
        core_axis_name="c", subcore_axis_name="s", num_cores=NC, num_subcores=NT
    )

    @functools.partial(
        pl.kernel,
        out_type=jax.ShapeDtypeStruct((NC * NPAD_DEG,), jnp.float32),
        mesh=mesh,
        scratch_types=[
            pltpu.VMEM((CHUNK,), jnp.int32),
            pltpu.VMEM((CHUNK,), jnp.float32),
            pltpu.VMEM_SHARED((NPAD_DEG,), jnp.float32),
        ],
    )
    def _sc_degree(dst_hbm, ones_hbm, zeros_hbm, out_hbm, didx, ones_v, accd):
        c = lax.axis_index("c")
        s = lax.axis_index("s")
        wid = c * NT + s

        pltpu.sync_copy(ones_hbm, ones_v)
        pltpu.sync_copy(zeros_hbm, accd.at[pl.ds(s * RPT_DEG, RPT_DEG)])
        plsc.subcore_barrier()

        ept = EPAD_DEG // (NC * NT)
        eb = wid * ept

        def body(i, carry):
            pltpu.sync_copy(dst_hbm.at[pl.ds(eb + i * CHUNK, CHUNK)], didx)
            pltpu.sync_copy(ones_v, accd.at[didx], add=True)
            return carry

        lax.fori_loop(0, ept // CHUNK, body, 0)
        plsc.subcore_barrier()
        pltpu.sync_copy(
            accd.at[pl.ds(s * RPT_DEG, RPT_DEG)],
            out_hbm.at[pl.ds(c * NPAD_DEG + s * RPT_DEG, RPT_DEG)],
        )

    @functools.partial(
        pl.kernel,
        out_type=jax.ShapeDtypeStruct((NC, NPAD, DH), jnp.float32),
        mesh=mesh,
        scratch_types=[
            pltpu.VMEM((2 * NBUF, CHUNK), jnp.int32),
            pltpu.VMEM((NBUF * CHUNK, DH), jnp.float32),
            pltpu.VMEM_SHARED((NPAD, DH), jnp.float32),
            pltpu.SemaphoreType.DMA,
            pltpu.SemaphoreType.DMA,
        ],
    )
    def _sc_aggregate(h_hbm, idx_hbm, zeros_hbm, out_hbm,
                      cidx, rows, acc, gsem, ssem):
        c = lax.axis_index("c")
        s = lax.axis_index("s")
        hview = h_hbm.at[c]

        pltpu.sync_copy(zeros_hbm, acc.at[pl.ds(s * RPT, RPT)])
        plsc.subcore_barrier()

        bpt = (EPAD // (NBUF * CHUNK)) // NT
        bb = s * bpt

        def group(g, carry):
            blk = bb + g
            pltpu.sync_copy(idx_hbm.at[blk], cidx)
            gds = [
                pltpu.async_copy(
                    hview.at[cidx.at[b]], rows.at[pl.ds(b * CHUNK, CHUNK)],
                    gsem)
                for b in range(NBUF)
            ]
            sds = []
            for b in range(NBUF):
                gds[b].wait()
                sds.append(
                    pltpu.async_copy(
                        rows.at[pl.ds(b * CHUNK, CHUNK)],
                        acc.at[cidx.at[NBUF + b]], ssem, add=True)
                )
            for d in sds:
                d.wait()
            return carry

        lax.fori_loop(0, bpt, group, 0)
        plsc.subcore_barrier()
        pltpu.sync_copy(
            acc.at[pl.ds(s * RPT, RPT)], out_hbm.at[c, pl.ds(s * RPT, RPT)]
        )

    return _sc_degree, _sc_aggregate


def _tc_head_body(x_ref, w_ref, deg_ref, o_ref):
    dinv = lax.rsqrt(deg_ref[...])
    h = jnp.dot(x_ref[...], w_ref[...], preferred_element_type=jnp.float32) * dinv
    o_ref[0] = h[:, :DH]
    o_ref[1] = h[:, DH:]


def _build_tc_head(interpret=False):
    return pl.pallas_call(
        _tc_head_body,
        grid=(NRB,),
        in_specs=[
            pl.BlockSpec((RB, D), lambda r: (r, 0)),
            pl.BlockSpec((D, D), lambda r: (0, 0)),
            pl.BlockSpec((RB, 1), lambda r: (r, 0)),
        ],
        out_specs=pl.BlockSpec((NC, RB, DH), lambda r: (0, r, 0)),
        out_shape=jax.ShapeDtypeStruct((NC, N, DH), jnp.float32),
        interpret=interpret,
    )


_tc_head = _build_tc_head()


def _layer_tail_y(s_ref, h_ref, deg_ref, b_ref):
    dinv = lax.rsqrt(deg_ref[...])
    sconc = jnp.concatenate([s_ref[0], s_ref[1]], axis=1)
    hconc = jnp.concatenate([h_ref[0], h_ref[1]], axis=1)
    y = (sconc + hconc) * dinv + b_ref[...]
    return y, dinv


def _tc_mid_body(s_ref, h_ref, deg_ref, b_ref, g_ref, be_ref, a_ref, w_ref,
                 o_ref, acc_ref):
    p = pl.program_id(0)
    r = pl.program_id(1)
    y, dinv = _layer_tail_y(s_ref, h_ref, deg_ref, b_ref)

    @pl.when(jnp.logical_and(p == 0, r == 0))
    def _():
        acc_ref[...] = jnp.zeros_like(acc_ref)

    @pl.when(p == 0)
    def _():
        acc_ref[0:1, :] += jnp.sum(y, axis=0, keepdims=True)
        acc_ref[1:2, :] += jnp.sum(y * y, axis=0, keepdims=True)
        o_ref[0] = y[:, :DH]
        o_ref[1] = y[:, DH:]

    @pl.when(p == 1)
    def _():
        mean = acc_ref[0:1, :] / N
        var = acc_ref[1:2, :] / N - mean * mean
        inv = lax.rsqrt(jnp.maximum(var, 0.0) + 1e-5)
        z = (y - mean) * inv * g_ref[...] + be_ref[...]
        z = jnp.where(z >= 0, z, z * a_ref[...])
        h2 = jnp.dot(z, w_ref[...], preferred_element_type=jnp.float32) * dinv
        o_ref[0] = h2[:, :DH]
        o_ref[1] = h2[:, DH:]


def _build_tc_mid(interpret=False):
    return pl.pallas_call(
        _tc_mid_body,
        grid=(2, NRB),
        in_specs=[
            pl.BlockSpec((NC, RB, DH), lambda p, r: (0, r, 0)),
            pl.BlockSpec((NC, RB, DH), lambda p, r: (0, r, 0)),
            pl.BlockSpec((RB, 1), lambda p, r: (r, 0)),
            pl.BlockSpec((1, D), lambda p, r: (0, 0)),
            pl.BlockSpec((1, D), lambda p, r: (0, 0)),
            pl.BlockSpec((1, D), lambda p, r: (0, 0)),
            pl.BlockSpec((1, 1), lambda p, r: (0, 0)),
            pl.BlockSpec((D, D), lambda p, r: (0, 0)),
        ],
        out_specs=pl.BlockSpec((NC, RB, DH), lambda p, r: (0, r, 0)),
        out_shape=jax.ShapeDtypeStruct((NC, N, DH), jnp.float32),
        scratch_shapes=[pltpu.VMEM((2, D), jnp.float32)],
        interpret=interpret,
    )


_tc_mid = _build_tc_mid()


def _tc_tail_body(s_ref, h_ref, deg_ref, b_ref, g_ref, be_ref, a_ref,
                  o_ref, acc_ref):
    p = pl.program_id(0)
    r = pl.program_id(1)
    y, _ = _layer_tail_y(s_ref, h_ref, deg_ref, b_ref)

    @pl.when(jnp.logical_and(p == 0, r == 0))
    def _():
        acc_ref[...] = jnp.zeros_like(acc_ref)

    @pl.when(p == 0)
    def _():
        acc_ref[0:1, :] += jnp.sum(y, axis=0, keepdims=True)
        acc_ref[1:2, :] += jnp.sum(y * y, axis=0, keepdims=True)
        o_ref[...] = y

    @pl.when(p == 1)
    def _():
        mean = acc_ref[0:1, :] / N
        var = acc_ref[1:2, :] / N - mean * mean
        inv = lax.rsqrt(jnp.maximum(var, 0.0) + 1e-5)
        z = (y - mean) * inv * g_ref[...] + be_ref[...]
        o_ref[...] = jnp.where(z >= 0, z, z * a_ref[...])


def _build_tc_tail(interpret=False):
    return pl.pallas_call(
        _tc_tail_body,
        grid=(2, NRB),
        in_specs=[
            pl.BlockSpec((NC, RB, DH), lambda p, r: (0, r, 0)),
            pl.BlockSpec((NC, RB, DH), lambda p, r: (0, r, 0)),
            pl.BlockSpec((RB, 1), lambda p, r: (r, 0)),
            pl.BlockSpec((1, D), lambda p, r: (0, 0)),
            pl.BlockSpec((1, D), lambda p, r: (0, 0)),
            pl.BlockSpec((1, D), lambda p, r: (0, 0)),
            pl.BlockSpec((1, 1), lambda p, r: (0, 0)),
        ],
        out_specs=pl.BlockSpec((RB, D), lambda p, r: (r, 0)),
        out_shape=jax.ShapeDtypeStruct((N, D), jnp.float32),
        scratch_shapes=[pltpu.VMEM((2, D), jnp.float32)],
        interpret=interpret,
    )


_tc_tail = _build_tc_tail()


def kernel(x, edge_index, W1, b1, g1, be1, a1, W2, b2, g2, be2, a2):
    src = edge_index[0]
    dst = edge_index[1]
    dst_pd = jnp.concatenate([dst, jnp.full((EPAD_DEG - E,), N, jnp.int32)])
    src_pa = jnp.concatenate([src, jnp.zeros((EPAD - E,), jnp.int32)])
    dst_pa = jnp.concatenate([dst, jnp.full((EPAD - E,), N, jnp.int32)])
    nblk = EPAD // (NBUF * CHUNK)
    comb = jnp.concatenate(
        [src_pa.reshape(nblk, NBUF, CHUNK), dst_pa.reshape(nblk, NBUF, CHUNK)],
        axis=1,
    )

    ones1 = jnp.ones((CHUNK,), jnp.float32)
    zeros1 = jnp.zeros((RPT_DEG,), jnp.float32)
    zeros_dh = jnp.zeros((RPT, DH), jnp.float32)

    b1r = b1.reshape(1, D)
    g1r = g1.reshape(1, D)
    be1r = be1.reshape(1, D)
    a1r = a1.reshape(1, 1)
    b2r = b2.reshape(1, D)
    g2r = g2.reshape(1, D)
    be2r = be2.reshape(1, D)
    a2r = a2.reshape(1, 1)

    sc_degree, sc_aggregate = _sc_kernels()
    degpair = sc_degree(dst_pd, ones1, zeros1)
    deg = (1.0 + degpair[:NPAD_DEG] + degpair[NPAD_DEG:]).reshape(NPAD_DEG, 1)

    h1 = _tc_head(x, W1, deg)
    s1 = sc_aggregate(h1, comb, zeros_dh)
    h2 = _tc_mid(s1, h1, deg, b1r, g1r, be1r, a1r, W2)
    s2 = sc_aggregate(h2, comb, zeros_dh)
    return _tc_tail(s2, h2, deg, b2r, g2r, be2r, a2r)


# speedup vs baseline: 1.2074x; 1.2074x over previous
import functools

import jax
import jax.numpy as jnp
from jax import lax
from jax.experimental import pallas as pl
from jax.experimental.pallas import tpu as pltpu
from jax.experimental.pallas import tpu_sc as plsc

N = 10000
E = 160000
D = 256
DH = 128

NC = 2
NT = 16
NPAD = 10240
RPT = NPAD // NT
EPAD = 163840
CHUNK = 128
NBUF = 2

RB = 1000
NRB = N // RB


@functools.cache
def _sc_kernels():
    mesh = plsc.VectorSubcoreMesh(
        core_axis_name="c", subcore_axis_name="s", num_cores=NC, num_subcores=NT
    )

    @functools.partial(
        pl.kernel,
        out_type=jax.ShapeDtypeStruct((NC * NPAD,), jnp.float32),
        mesh=mesh,
        scratch_types=[
            pltpu.VMEM((CHUNK,), jnp.int32),
            pltpu.VMEM((CHUNK,), jnp.float32),
            pltpu.VMEM_SHARED((NPAD,), jnp.float32),
        ],
    )
    def _sc_degree(dst_hbm, ones_hbm, zeros_hbm, out_hbm, didx, ones_v, accd):
        c = lax.axis_index("c")
        s = lax.axis_index("s")
        wid = c * NT + s

        pltpu.sync_copy(ones_hbm, ones_v)
        pltpu.sync_copy(zeros_hbm, accd.at[pl.ds(s * RPT, RPT)])
        plsc.subcore_barrier()

        ept = EPAD // (NC * NT)
        eb = wid * ept

        def body(i, carry):
            pltpu.sync_copy(dst_hbm.at[pl.ds(eb + i * CHUNK, CHUNK)], didx)
            pltpu.sync_copy(ones_v, accd.at[didx], add=True)
            return carry

        lax.fori_loop(0, ept // CHUNK, body, 0)
        plsc.subcore_barrier()
        pltpu.sync_copy(
            accd.at[pl.ds(s * RPT, RPT)], out_hbm.at[pl.ds(c * NPAD + s * RPT, RPT)]
        )

    @functools.partial(
        pl.kernel,
        out_type=jax.ShapeDtypeStruct((NC, NPAD, DH), jnp.float32),
        mesh=mesh,
        scratch_types=[
            pltpu.VMEM((2 * NBUF, CHUNK), jnp.int32),
            pltpu.VMEM((NBUF * CHUNK, DH), jnp.float32),
            pltpu.VMEM_SHARED((NPAD, DH), jnp.float32),
            pltpu.SemaphoreType.DMA,
            pltpu.SemaphoreType.DMA,
        ],
    )
    def _sc_aggregate(h_hbm, idx_hbm, zeros_hbm, out_hbm,
                      cidx, rows, acc, gsem, ssem):
        c = lax.axis_index("c")
        s = lax.axis_index("s")
        hview = h_hbm.at[c]

        pltpu.sync_copy(zeros_hbm, acc.at[pl.ds(s * RPT, RPT)])
        plsc.subcore_barrier()

        bpt = (EPAD // (NBUF * CHUNK)) // NT
        bb = s * bpt

        def group(g, carry):
            blk = bb + g
            pltpu.sync_copy(idx_hbm.at[blk], cidx)
            gds = [
                pltpu.async_copy(
                    hview.at[cidx.at[b]], rows.at[pl.ds(b * CHUNK, CHUNK)],
                    gsem)
                for b in range(NBUF)
            ]
            sds = []
            for b in range(NBUF):
                gds[b].wait()
                sds.append(
                    pltpu.async_copy(
                        rows.at[pl.ds(b * CHUNK, CHUNK)], acc.at[cidx.at[NBUF + b]],
                        ssem, add=True)
                )
            for d in sds:
                d.wait()
            return carry

        lax.fori_loop(0, bpt, group, 0)
        plsc.subcore_barrier()
        pltpu.sync_copy(
            acc.at[pl.ds(s * RPT, RPT)], out_hbm.at[c, pl.ds(s * RPT, RPT)]
        )

    return _sc_degree, _sc_aggregate


def _tc_head_body(x_ref, w_ref, deg_ref, o_ref):
    dinv = lax.rsqrt(deg_ref[...])
    h = jnp.dot(x_ref[...], w_ref[...], preferred_element_type=jnp.float32) * dinv
    o_ref[0] = h[:, :DH]
    o_ref[1] = h[:, DH:]


def _build_tc_head(interpret=False):
    return pl.pallas_call(
        _tc_head_body,
        grid=(NRB,),
        in_specs=[
            pl.BlockSpec((RB, D), lambda r: (r, 0)),
            pl.BlockSpec((D, D), lambda r: (0, 0)),
            pl.BlockSpec((RB, 1), lambda r: (r, 0)),
        ],
        out_specs=pl.BlockSpec((NC, RB, DH), lambda r: (0, r, 0)),
        out_shape=jax.ShapeDtypeStruct((NC, N, DH), jnp.float32),
        interpret=interpret,
    )


_tc_head = _build_tc_head()


def _layer_tail_y(s_ref, h_ref, deg_ref, b_ref):
    dinv = lax.rsqrt(deg_ref[...])
    sconc = jnp.concatenate([s_ref[0], s_ref[1]], axis=1)
    hconc = jnp.concatenate([h_ref[0], h_ref[1]], axis=1)
    y = (sconc + hconc) * dinv + b_ref[...]
    return y, dinv


def _tc_mid_body(s_ref, h_ref, deg_ref, b_ref, g_ref, be_ref, a_ref, w_ref,
                 o_ref, acc_ref):
    p = pl.program_id(0)
    r = pl.program_id(1)
    y, dinv = _layer_tail_y(s_ref, h_ref, deg_ref, b_ref)

    @pl.when(jnp.logical_and(p == 0, r == 0))
    def _():
        acc_ref[...] = jnp.zeros_like(acc_ref)

    @pl.when(p == 0)
    def _():
        acc_ref[0:1, :] += jnp.sum(y, axis=0, keepdims=True)
        acc_ref[1:2, :] += jnp.sum(y * y, axis=0, keepdims=True)
        o_ref[0] = y[:, :DH]
        o_ref[1] = y[:, DH:]

    @pl.when(p == 1)
    def _():
        mean = acc_ref[0:1, :] / N
        var = acc_ref[1:2, :] / N - mean * mean
        inv = lax.rsqrt(jnp.maximum(var, 0.0) + 1e-5)
        z = (y - mean) * inv * g_ref[...] + be_ref[...]
        z = jnp.where(z >= 0, z, z * a_ref[...])
        h2 = jnp.dot(z, w_ref[...], preferred_element_type=jnp.float32) * dinv
        o_ref[0] = h2[:, :DH]
        o_ref[1] = h2[:, DH:]


def _build_tc_mid(interpret=False):
    return pl.pallas_call(
        _tc_mid_body,
        grid=(2, NRB),
        in_specs=[
            pl.BlockSpec((NC, RB, DH), lambda p, r: (0, r, 0)),
            pl.BlockSpec((NC, RB, DH), lambda p, r: (0, r, 0)),
            pl.BlockSpec((RB, 1), lambda p, r: (r, 0)),
            pl.BlockSpec((1, D), lambda p, r: (0, 0)),
            pl.BlockSpec((1, D), lambda p, r: (0, 0)),
            pl.BlockSpec((1, D), lambda p, r: (0, 0)),
            pl.BlockSpec((1, 1), lambda p, r: (0, 0)),
            pl.BlockSpec((D, D), lambda p, r: (0, 0)),
        ],
        out_specs=pl.BlockSpec((NC, RB, DH), lambda p, r: (0, r, 0)),
        out_shape=jax.ShapeDtypeStruct((NC, N, DH), jnp.float32),
        scratch_shapes=[pltpu.VMEM((2, D), jnp.float32)],
        interpret=interpret,
    )


_tc_mid = _build_tc_mid()


def _tc_tail_body(s_ref, h_ref, deg_ref, b_ref, g_ref, be_ref, a_ref,
                  o_ref, acc_ref):
    p = pl.program_id(0)
    r = pl.program_id(1)
    y, _ = _layer_tail_y(s_ref, h_ref, deg_ref, b_ref)

    @pl.when(jnp.logical_and(p == 0, r == 0))
    def _():
        acc_ref[...] = jnp.zeros_like(acc_ref)

    @pl.when(p == 0)
    def _():
        acc_ref[0:1, :] += jnp.sum(y, axis=0, keepdims=True)
        acc_ref[1:2, :] += jnp.sum(y * y, axis=0, keepdims=True)
        o_ref[...] = y

    @pl.when(p == 1)
    def _():
        mean = acc_ref[0:1, :] / N
        var = acc_ref[1:2, :] / N - mean * mean
        inv = lax.rsqrt(jnp.maximum(var, 0.0) + 1e-5)
        z = (y - mean) * inv * g_ref[...] + be_ref[...]
        o_ref[...] = jnp.where(z >= 0, z, z * a_ref[...])


def _build_tc_tail(interpret=False):
    return pl.pallas_call(
        _tc_tail_body,
        grid=(2, NRB),
        in_specs=[
            pl.BlockSpec((NC, RB, DH), lambda p, r: (0, r, 0)),
            pl.BlockSpec((NC, RB, DH), lambda p, r: (0, r, 0)),
            pl.BlockSpec((RB, 1), lambda p, r: (r, 0)),
            pl.BlockSpec((1, D), lambda p, r: (0, 0)),
            pl.BlockSpec((1, D), lambda p, r: (0, 0)),
            pl.BlockSpec((1, D), lambda p, r: (0, 0)),
            pl.BlockSpec((1, 1), lambda p, r: (0, 0)),
        ],
        out_specs=pl.BlockSpec((RB, D), lambda p, r: (r, 0)),
        out_shape=jax.ShapeDtypeStruct((N, D), jnp.float32),
        scratch_shapes=[pltpu.VMEM((2, D), jnp.float32)],
        interpret=interpret,
    )


_tc_tail = _build_tc_tail()


def kernel(x, edge_index, W1, b1, g1, be1, a1, W2, b2, g2, be2, a2):
    src = edge_index[0]
    dst = edge_index[1]
    pad = EPAD - E
    src_p = jnp.concatenate([src, jnp.zeros((pad,), jnp.int32)])
    dst_p = jnp.concatenate([dst, jnp.full((pad,), N, jnp.int32)])

    ones1 = jnp.ones((CHUNK,), jnp.float32)
    zeros1 = jnp.zeros((RPT,), jnp.float32)
    zeros_dh = jnp.zeros((RPT, DH), jnp.float32)

    b1r = b1.reshape(1, D)
    g1r = g1.reshape(1, D)
    be1r = be1.reshape(1, D)
    a1r = a1.reshape(1, 1)
    b2r = b2.reshape(1, D)
    g2r = g2.reshape(1, D)
    be2r = be2.reshape(1, D)
    a2r = a2.reshape(1, 1)

    sc_degree, sc_aggregate = _sc_kernels()
    degpair = sc_degree(dst_p, ones1, zeros1)
    deg = (1.0 + degpair[:NPAD] + degpair[NPAD:]).reshape(NPAD, 1)

    nblk = EPAD // (NBUF * CHUNK)
    comb = jnp.concatenate(
        [src_p.reshape(nblk, NBUF, CHUNK), dst_p.reshape(nblk, NBUF, CHUNK)],
        axis=1,
    )
    h1 = _tc_head(x, W1, deg)
    s1 = sc_aggregate(h1, comb, zeros_dh)
    h2 = _tc_mid(s1, h1, deg, b1r, g1r, be1r, a1r, W2)
    s2 = sc_aggregate(h2, comb, zeros_dh)
    return _tc_tail(s2, h2, deg, b2r, g2r, be2r, a2r)


# speedup vs baseline: 1.2476x; 1.0333x over previous
import functools

import jax
import jax.numpy as jnp
from jax import lax
from jax.experimental import pallas as pl
from jax.experimental.pallas import tpu as pltpu
from jax.experimental.pallas import tpu_sc as plsc

N = 10000
E = 160000
D = 256
DH = 128

NC = 2
NT = 16
NPAD = 10240
RPT = NPAD // NT
EPAD = 163840
CHUNK = 128
NBUF = 2

RB = 1000
NRB = N // RB


@functools.cache
def _sc_kernels():
    mesh = plsc.VectorSubcoreMesh(
        core_axis_name="c", subcore_axis_name="s", num_cores=NC, num_subcores=NT
    )

    @functools.partial(
        pl.kernel,
        out_type=jax.ShapeDtypeStruct((NC * NPAD,), jnp.float32),
        mesh=mesh,
        scratch_types=[
            pltpu.VMEM((CHUNK,), jnp.int32),
            pltpu.VMEM((CHUNK,), jnp.float32),
            pltpu.VMEM_SHARED((NPAD,), jnp.float32),
        ],
    )
    def _sc_degree(dst_hbm, ones_hbm, zeros_hbm, out_hbm, didx, ones_v, accd):
        c = lax.axis_index("c")
        s = lax.axis_index("s")
        wid = c * NT + s

        pltpu.sync_copy(ones_hbm, ones_v)
        pltpu.sync_copy(zeros_hbm, accd.at[pl.ds(s * RPT, RPT)])
        plsc.subcore_barrier()

        ept = EPAD // (NC * NT)
        eb = wid * ept

        def body(i, carry):
            pltpu.sync_copy(dst_hbm.at[pl.ds(eb + i * CHUNK, CHUNK)], didx)
            pltpu.sync_copy(ones_v, accd.at[didx], add=True)
            return carry

        lax.fori_loop(0, ept // CHUNK, body, 0)
        plsc.subcore_barrier()
        pltpu.sync_copy(
            accd.at[pl.ds(s * RPT, RPT)], out_hbm.at[pl.ds(c * NPAD + s * RPT, RPT)]
        )

    @functools.partial(
        pl.kernel,
        out_type=jax.ShapeDtypeStruct((NC, NPAD, DH), jnp.float32),
        mesh=mesh,
        scratch_types=[
            pltpu.VMEM((2, 2 * NBUF, CHUNK), jnp.int32),
            pltpu.VMEM((NBUF * CHUNK, DH), jnp.float32),
            pltpu.VMEM_SHARED((NPAD, DH), jnp.float32),
            pltpu.SemaphoreType.DMA,
            pltpu.SemaphoreType.DMA,
            pltpu.SemaphoreType.DMA,
        ],
    )
    def _sc_aggregate(h_hbm, idx_hbm, zeros_hbm, out_hbm,
                      cidx, rows, acc, gsem, ssem, isem):
        c = lax.axis_index("c")
        s = lax.axis_index("s")
        hview = h_hbm.at[c]

        pltpu.sync_copy(zeros_hbm, acc.at[pl.ds(s * RPT, RPT)])
        plsc.subcore_barrier()

        bpt = (EPAD // (NBUF * CHUNK)) // NT
        bb = s * bpt

        pltpu.sync_copy(idx_hbm.at[bb], cidx.at[0])
        pltpu.async_copy(idx_hbm.at[bb + 1], cidx.at[1], isem)

        def do_group(idxv):
            gds = [
                pltpu.async_copy(
                    hview.at[idxv.at[b]], rows.at[pl.ds(b * CHUNK, CHUNK)],
                    gsem)
                for b in range(NBUF)
            ]
            sds = []
            for b in range(NBUF):
                gds[b].wait()
                sds.append(
                    pltpu.async_copy(
                        rows.at[pl.ds(b * CHUNK, CHUNK)],
                        acc.at[idxv.at[NBUF + b]], ssem, add=True)
                )
            for d in sds:
                d.wait()

        def pair(i, carry):
            blk = bb + 2 * i
            for par in (0, 1):
                do_group(cidx.at[par])
                nxt = blk + par + 2

                @pl.when(nxt < bb + bpt)
                def _():
                    pltpu.async_copy(idx_hbm.at[nxt], cidx.at[par], isem)

                @pl.when(nxt - 1 < bb + bpt)
                def _():
                    pltpu.make_async_copy(
                        idx_hbm.at[nxt - 1], cidx.at[1 - par], isem
                    ).wait()
            return carry

        lax.fori_loop(0, bpt // 2, pair, 0)
        plsc.subcore_barrier()
        pltpu.sync_copy(
            acc.at[pl.ds(s * RPT, RPT)], out_hbm.at[c, pl.ds(s * RPT, RPT)]
        )

    return _sc_degree, _sc_aggregate


def _tc_head_body(x_ref, w_ref, deg_ref, o_ref):
    dinv = lax.rsqrt(deg_ref[...])
    h = jnp.dot(x_ref[...], w_ref[...], preferred_element_type=jnp.float32) * dinv
    o_ref[0] = h[:, :DH]
    o_ref[1] = h[:, DH:]


def _build_tc_head(interpret=False):
    return pl.pallas_call(
        _tc_head_body,
        grid=(NRB,),
        in_specs=[
            pl.BlockSpec((RB, D), lambda r: (r, 0)),
            pl.BlockSpec((D, D), lambda r: (0, 0)),
            pl.BlockSpec((RB, 1), lambda r: (r, 0)),
        ],
        out_specs=pl.BlockSpec((NC, RB, DH), lambda r: (0, r, 0)),
        out_shape=jax.ShapeDtypeStruct((NC, N, DH), jnp.float32),
        interpret=interpret,
    )


_tc_head = _build_tc_head()


def _layer_tail_y(s_ref, h_ref, deg_ref, b_ref):
    dinv = lax.rsqrt(deg_ref[...])
    sconc = jnp.concatenate([s_ref[0], s_ref[1]], axis=1)
    hconc = jnp.concatenate([h_ref[0], h_ref[1]], axis=1)
    y = (sconc + hconc) * dinv + b_ref[...]
    return y, dinv


def _tc_mid_body(s_ref, h_ref, deg_ref, b_ref, g_ref, be_ref, a_ref, w_ref,
                 o_ref, acc_ref):
    p = pl.program_id(0)
    r = pl.program_id(1)
    y, dinv = _layer_tail_y(s_ref, h_ref, deg_ref, b_ref)

    @pl.when(jnp.logical_and(p == 0, r == 0))
    def _():
        acc_ref[...] = jnp.zeros_like(acc_ref)

    @pl.when(p == 0)
    def _():
        acc_ref[0:1, :] += jnp.sum(y, axis=0, keepdims=True)
        acc_ref[1:2, :] += jnp.sum(y * y, axis=0, keepdims=True)
        o_ref[0] = y[:, :DH]
        o_ref[1] = y[:, DH:]

    @pl.when(p == 1)
    def _():
        mean = acc_ref[0:1, :] / N
        var = acc_ref[1:2, :] / N - mean * mean
        inv = lax.rsqrt(jnp.maximum(var, 0.0) + 1e-5)
        z = (y - mean) * inv * g_ref[...] + be_ref[...]
        z = jnp.where(z >= 0, z, z * a_ref[...])
        h2 = jnp.dot(z, w_ref[...], preferred_element_type=jnp.float32) * dinv
        o_ref[0] = h2[:, :DH]
        o_ref[1] = h2[:, DH:]


def _build_tc_mid(interpret=False):
    return pl.pallas_call(
        _tc_mid_body,
        grid=(2, NRB),
        in_specs=[
            pl.BlockSpec((NC, RB, DH), lambda p, r: (0, r, 0)),
            pl.BlockSpec((NC, RB, DH), lambda p, r: (0, r, 0)),
            pl.BlockSpec((RB, 1), lambda p, r: (r, 0)),
            pl.BlockSpec((1, D), lambda p, r: (0, 0)),
            pl.BlockSpec((1, D), lambda p, r: (0, 0)),
            pl.BlockSpec((1, D), lambda p, r: (0, 0)),
            pl.BlockSpec((1, 1), lambda p, r: (0, 0)),
            pl.BlockSpec((D, D), lambda p, r: (0, 0)),
        ],
        out_specs=pl.BlockSpec((NC, RB, DH), lambda p, r: (0, r, 0)),
        out_shape=jax.ShapeDtypeStruct((NC, N, DH), jnp.float32),
        scratch_shapes=[pltpu.VMEM((2, D), jnp.float32)],
        interpret=interpret,
    )


_tc_mid = _build_tc_mid()


def _tc_tail_body(s_ref, h_ref, deg_ref, b_ref, g_ref, be_ref, a_ref,
                  o_ref, acc_ref):
    p = pl.program_id(0)
    r = pl.program_id(1)
    y, _ = _layer_tail_y(s_ref, h_ref, deg_ref, b_ref)

    @pl.when(jnp.logical_and(p == 0, r == 0))
    def _():
        acc_ref[...] = jnp.zeros_like(acc_ref)

    @pl.when(p == 0)
    def _():
        acc_ref[0:1, :] += jnp.sum(y, axis=0, keepdims=True)
        acc_ref[1:2, :] += jnp.sum(y * y, axis=0, keepdims=True)
        o_ref[...] = y

    @pl.when(p == 1)
    def _():
        mean = acc_ref[0:1, :] / N
        var = acc_ref[1:2, :] / N - mean * mean
        inv = lax.rsqrt(jnp.maximum(var, 0.0) + 1e-5)
        z = (y - mean) * inv * g_ref[...] + be_ref[...]
        o_ref[...] = jnp.where(z >= 0, z, z * a_ref[...])


def _build_tc_tail(interpret=False):
    return pl.pallas_call(
        _tc_tail_body,
        grid=(2, NRB),
        in_specs=[
            pl.BlockSpec((NC, RB, DH), lambda p, r: (0, r, 0)),
            pl.BlockSpec((NC, RB, DH), lambda p, r: (0, r, 0)),
            pl.BlockSpec((RB, 1), lambda p, r: (r, 0)),
            pl.BlockSpec((1, D), lambda p, r: (0, 0)),
            pl.BlockSpec((1, D), lambda p, r: (0, 0)),
            pl.BlockSpec((1, D), lambda p, r: (0, 0)),
            pl.BlockSpec((1, 1), lambda p, r: (0, 0)),
        ],
        out_specs=pl.BlockSpec((RB, D), lambda p, r: (r, 0)),
        out_shape=jax.ShapeDtypeStruct((N, D), jnp.float32),
        scratch_shapes=[pltpu.VMEM((2, D), jnp.float32)],
        interpret=interpret,
    )


_tc_tail = _build_tc_tail()


def kernel(x, edge_index, W1, b1, g1, be1, a1, W2, b2, g2, be2, a2):
    src = edge_index[0]
    dst = edge_index[1]
    pad = EPAD - E
    src_p = jnp.concatenate([src, jnp.zeros((pad,), jnp.int32)])
    dst_p = jnp.concatenate([dst, jnp.full((pad,), N, jnp.int32)])

    ones1 = jnp.ones((CHUNK,), jnp.float32)
    zeros1 = jnp.zeros((RPT,), jnp.float32)
    zeros_dh = jnp.zeros((RPT, DH), jnp.float32)

    b1r = b1.reshape(1, D)
    g1r = g1.reshape(1, D)
    be1r = be1.reshape(1, D)
    a1r = a1.reshape(1, 1)
    b2r = b2.reshape(1, D)
    g2r = g2.reshape(1, D)
    be2r = be2.reshape(1, D)
    a2r = a2.reshape(1, 1)

    sc_degree, sc_aggregate = _sc_kernels()
    degpair = sc_degree(dst_p, ones1, zeros1)
    deg = (1.0 + degpair[:NPAD] + degpair[NPAD:]).reshape(NPAD, 1)

    nblk = EPAD // (NBUF * CHUNK)
    comb = jnp.concatenate(
        [src_p.reshape(nblk, NBUF, CHUNK), dst_p.reshape(nblk, NBUF, CHUNK)],
        axis=1,
    )
    h1 = _tc_head(x, W1, deg)
    s1 = sc_aggregate(h1, comb, zeros_dh)
    h2 = _tc_mid(s1, h1, deg, b1r, g1r, be1r, a1r, W2)
    s2 = sc_aggregate(h2, comb, zeros_dh)
    return _tc_tail(s2, h2, deg, b2r, g2r, be2r, a2r)
